# Initial kernel scaffold; baseline (speedup 1.0000x reference)
import functools
import math

import jax
import jax.numpy as jnp
from jax import lax
from jax.experimental import pallas as pl
from jax.experimental.pallas import tpu as pltpu
from jax.experimental.pallas import tpu_sc as plsc

D = 128
COEFF = math.sqrt(128.0)
LANES = 16

NC, NS = 2, 16
NW = NC * NS
B = 4096 * 50
BPW = B // NW
CHUNK = 256
GSUB = CHUNK // 128
NCHUNK = BPW // CHUNK

_mesh = plsc.VectorSubcoreMesh(core_axis_name="c", subcore_axis_name="s")


@functools.partial(
    pl.kernel,
    mesh=_mesh,
    out_type=jax.ShapeDtypeStruct((B, D), jnp.float32),
    scratch_types=[
        pltpu.VMEM((GSUB, 128), jnp.int32),
        pltpu.VMEM((CHUNK, D), jnp.float32),
        pltpu.SemaphoreType.DMA,
    ],
)
def _emb_lookup(table_hbm, idx_hbm, out_hbm, idx_v, rows_v, sem):
    wid = lax.axis_index("s") * NC + lax.axis_index("c")
    base = wid * BPW

    def chunk_body(g, carry):
        off = base + g * CHUNK
        pltpu.sync_copy(idx_hbm.at[pl.ds(off // 128, GSUB)], idx_v)
        copies = []
        for j in range(GSUB):
            copies.append(
                pltpu.async_copy(
                    table_hbm.at[idx_v.at[j]],
                    rows_v.at[pl.ds(j * 128, 128)],
                    sem,
                )
            )
        for c in copies:
            c.wait()

        def row_body(i, c):
            for j in range(D // LANES):
                sl = pl.ds(j * LANES, LANES)
                rows_v[i, sl] = rows_v[i, sl] * COEFF
            return c

        lax.fori_loop(0, CHUNK, row_body, 0, unroll=False)

        pltpu.sync_copy(rows_v, out_hbm.at[pl.ds(off, CHUNK)])
        return carry

    lax.fori_loop(0, NCHUNK, chunk_body, 0, unroll=False)


def kernel(x, table):
    idx = x.reshape(B // 128, 128).astype(jnp.int32)
    out = _emb_lookup(table, idx)
    return out.reshape(x.shape[0], x.shape[1], D)


# speedup vs baseline: 2.4816x; 2.4816x over previous
import functools
import math

import jax
import jax.numpy as jnp
from jax import lax
from jax.experimental import pallas as pl
from jax.experimental.pallas import tpu as pltpu
from jax.experimental.pallas import tpu_sc as plsc

D = 128
COEFF = math.sqrt(128.0)
LANES = 16

NC, NS = 2, 16
NW = NC * NS
B = 4096 * 50
BPW = B // NW
CHUNK = 256
GSUB = CHUNK // 128
NCHUNK = BPW // CHUNK

_mesh = plsc.VectorSubcoreMesh(core_axis_name="c", subcore_axis_name="s")


@functools.partial(
    pl.kernel,
    mesh=_mesh,
    out_type=jax.ShapeDtypeStruct((B, D), jnp.float32),
    scratch_types=[
        pltpu.VMEM((CHUNK,), jnp.int32),
        pltpu.VMEM((CHUNK, D), jnp.float32),
        pltpu.SemaphoreType.DMA,
    ],
)
def _emb_lookup(table_hbm, idx_hbm, out_hbm, idx_v, rows_v, sem):
    wid = lax.axis_index("s") * NC + lax.axis_index("c")
    base = wid * BPW

    def chunk_body(g, carry):
        off = base + g * CHUNK
        pltpu.sync_copy(idx_hbm.at[pl.ds(off, CHUNK)], idx_v)
        copies = []
        for j in range(GSUB):
            copies.append(
                pltpu.async_copy(
                    table_hbm.at[idx_v.at[pl.ds(j * 128, 128)]],
                    rows_v.at[pl.ds(j * 128, 128)],
                    sem,
                )
            )
        for c in copies:
            c.wait()

        def row_body(i, c):
            for j in range(D // LANES):
                sl = pl.ds(j * LANES, LANES)
                rows_v[i, sl] = rows_v[i, sl] * COEFF
            return c

        lax.fori_loop(0, CHUNK, row_body, 0, unroll=False)

        pltpu.sync_copy(rows_v, out_hbm.at[pl.ds(off, CHUNK)])
        return carry

    lax.fori_loop(0, NCHUNK, chunk_body, 0, unroll=False)


def kernel(x, table):
    idx = x.reshape(B).astype(jnp.int32)
    out = _emb_lookup(table, idx)
    return out.reshape(x.shape[0], x.shape[1], D)


# speedup vs baseline: 2.9526x; 1.1898x over previous
import functools
import math

import jax
import jax.numpy as jnp
from jax import lax
from jax.experimental import pallas as pl
from jax.experimental.pallas import tpu as pltpu
from jax.experimental.pallas import tpu_sc as plsc

D = 128
COEFF = math.sqrt(128.0)
LANES = 16

NC, NS = 2, 16
NW = NC * NS
B = 4096 * 50
BPW = B // NW
CHUNK = 128
NB = 5
NCHUNK = BPW // CHUNK
TGROUP = NCHUNK // NB

_mesh = plsc.VectorSubcoreMesh(core_axis_name="c", subcore_axis_name="s")


@functools.partial(
    pl.kernel,
    mesh=_mesh,
    out_type=jax.ShapeDtypeStruct((B, D), jnp.float32),
    scratch_types=(
        [pltpu.VMEM((BPW,), jnp.int32)]
        + [pltpu.VMEM((CHUNK, D), jnp.float32) for _ in range(NB)]
        + [pltpu.SemaphoreType.DMA for _ in range(2 * NB)]
    ),
)
def _emb_lookup(table_hbm, idx_hbm, out_hbm, idx_v, *bufs_and_sems):
    bufs = bufs_and_sems[:NB]
    gsem = bufs_and_sems[NB:2 * NB]
    osem = bufs_and_sems[2 * NB:]

    wid = lax.axis_index("s") * NC + lax.axis_index("c")
    base = wid * BPW

    pltpu.sync_copy(idx_hbm.at[pl.ds(base, BPW)], idx_v)

    def gather(g, b):
        pltpu.make_async_copy(
            table_hbm.at[idx_v.at[pl.ds(g * CHUNK, CHUNK)]],
            bufs[b],
            gsem[b],
        ).start()

    def wait_gather(b):
        pltpu.make_async_copy(
            table_hbm.at[idx_v.at[pl.ds(0, CHUNK)]], bufs[b], gsem[b]
        ).wait()

    def put(g, b):
        pltpu.make_async_copy(
            bufs[b], out_hbm.at[pl.ds(base + g * CHUNK, CHUNK)], osem[b]
        ).start()

    def wait_put(b):
        pltpu.make_async_copy(
            bufs[b], out_hbm.at[pl.ds(base, CHUNK)], osem[b]
        ).wait()

    for b in range(NB - 1):
        gather(b, b)

    def body(t, carry):
        for k in range(NB):
            b = k
            bn = (k + NB - 1) % NB
            g = t * NB + k
            @pl.when(g + (NB - 1) < NCHUNK)
            def _issue():
                @pl.when(g >= 1)
                def _drain():
                    wait_put(bn)
                gather(g + (NB - 1), bn)

            wait_gather(b)

            def row_body(i, c):
                for j in range(D // LANES):
                    sl = pl.ds(j * LANES, LANES)
                    bufs[b][i, sl] = bufs[b][i, sl] * COEFF
                return c

            lax.fori_loop(0, CHUNK, row_body, 0, unroll=False)
            put(g, b)
        return carry

    lax.fori_loop(0, TGROUP, body, 0, unroll=False)

    for b in range(NB):
        wait_put(b)


def kernel(x, table):
    idx = x.reshape(B).astype(jnp.int32)
    out = _emb_lookup(table, idx)
    return out.reshape(x.shape[0], x.shape[1], D)


# speedup vs baseline: 9.1657x; 3.1043x over previous
import functools
import math

import jax
import jax.numpy as jnp
from jax import lax
from jax.experimental import pallas as pl
from jax.experimental.pallas import tpu as pltpu
from jax.experimental.pallas import tpu_sc as plsc

D = 128
COEFF = math.sqrt(128.0)
LANES = 16

NC, NS = 2, 16
NW = NC * NS
B = 4096 * 50
BPW = B // NW
CHUNK = 128
NB = 5
NCHUNK = BPW // CHUNK
TGROUP = NCHUNK // NB

_mesh = plsc.VectorSubcoreMesh(core_axis_name="c", subcore_axis_name="s")


@functools.partial(
    pl.kernel,
    mesh=_mesh,
    out_type=jax.ShapeDtypeStruct((B, D), jnp.float32),
    scratch_types=(
        [pltpu.VMEM((BPW,), jnp.int32)]
        + [pltpu.VMEM((CHUNK, D), jnp.float32) for _ in range(NB)]
        + [pltpu.SemaphoreType.DMA for _ in range(2 * NB)]
    ),
)
def _emb_lookup(table_hbm, idx_hbm, out_hbm, idx_v, *bufs_and_sems):
    bufs = bufs_and_sems[:NB]
    gsem = bufs_and_sems[NB:2 * NB]
    osem = bufs_and_sems[2 * NB:]

    wid = lax.axis_index("s") * NC + lax.axis_index("c")
    base = wid * BPW

    pltpu.sync_copy(idx_hbm.at[pl.ds(base, BPW)], idx_v)

    def gather(g, b):
        pltpu.make_async_copy(
            table_hbm.at[idx_v.at[pl.ds(g * CHUNK, CHUNK)]],
            bufs[b],
            gsem[b],
        ).start()

    def wait_gather(b):
        pltpu.make_async_copy(
            table_hbm.at[idx_v.at[pl.ds(0, CHUNK)]], bufs[b], gsem[b]
        ).wait()

    def put(g, b):
        pltpu.make_async_copy(
            bufs[b], out_hbm.at[pl.ds(base + g * CHUNK, CHUNK)], osem[b]
        ).start()

    def wait_put(b):
        pltpu.make_async_copy(
            bufs[b], out_hbm.at[pl.ds(base, CHUNK)], osem[b]
        ).wait()

    for b in range(NB - 1):
        gather(b, b)

    def body(t, carry):
        for k in range(NB):
            b = k
            bn = (k + NB - 1) % NB
            g = t * NB + k
            @pl.when(g + (NB - 1) < NCHUNK)
            def _issue():
                @pl.when(g >= 1)
                def _drain():
                    wait_put(bn)
                gather(g + (NB - 1), bn)

            wait_gather(b)

            def row_body(i, c):
                for j in range(D // LANES):
                    sl = pl.ds(j * LANES, LANES)
                    bufs[b][i, sl] = bufs[b][i, sl] * COEFF
                return c

            lax.fori_loop(0, CHUNK, row_body, 0, unroll=False)
            put(g, b)
        return carry

    lax.fori_loop(0, TGROUP, body, 0, unroll=False)

    for b in range(NB):
        wait_put(b)


def kernel(x, table):
    idx = x.astype(jnp.int32).T.reshape(B)
    out = _emb_lookup(table, idx)
    return out.reshape(x.shape[1], x.shape[0], D).transpose(1, 0, 2)
